# baseline (speedup 1.0000x reference)
import functools

import jax
import jax.numpy as jnp
from jax.experimental import pallas as pl
from jax.experimental.pallas import tpu as pltpu

N = 100000
D = 128
H = 64
B = 64
BLK = 2000
NB = N // BLK


def _fused_body(x_ref, b_ref, wc_ref, w3c_ref, out_ref, zacc, dacc):
    i = pl.program_id(0)

    @pl.when(i == 0)
    def _init():
        zacc[:, :] = jnp.zeros_like(zacc)
        dacc[:, :] = jnp.zeros_like(dacc)

    f32 = jnp.float32
    bf16 = jnp.bfloat16
    xh = x_ref[:, :].astype(bf16)
    y = jax.lax.dot_general(
        xh, wc_ref[:, :], (((1,), (0,)), ((), ())),
        preferred_element_type=f32)
    u = jnp.tanh(y[:, :H])
    e = jnp.exp(y[:, H:])
    v = e / jnp.sum(e, axis=1, keepdims=True)
    g = (u * v).astype(bf16)
    alpha = jax.lax.dot_general(
        g, w3c_ref[:, :], (((1,), (0,)), ((), ())),
        preferred_element_type=f32)
    w = jnp.exp(alpha)

    ids = b_ref[0]
    seg = jax.lax.broadcasted_iota(jnp.int32, (BLK, B), 1)
    m = jnp.where(ids == seg, w, 0.0)

    zacc[:, :] += jax.lax.dot_general(
        m, x_ref[:, :], (((0,), (0,)), ((), ())),
        preferred_element_type=f32)
    dacc[:, :] += jax.lax.dot_general(
        m, jnp.ones((BLK, 1), f32), (((0,), (0,)), ((), ())),
        preferred_element_type=f32)

    @pl.when(i == NB - 1)
    def _emit():
        out_ref[:, :] = zacc[:, :] / jnp.maximum(dacc[:, :], 1e-30)


@functools.partial(jax.jit, static_argnames=("interpret",))
def _run(x, batch3, wc, w3c, interpret=False):
    return pl.pallas_call(
        _fused_body,
        grid=(NB,),
        in_specs=[
            pl.BlockSpec((BLK, D), lambda i: (i, 0)),
            pl.BlockSpec((1, BLK, 1), lambda i: (i, 0, 0)),
            pl.BlockSpec((D, 2 * H), lambda i: (0, 0)),
            pl.BlockSpec((H, 1), lambda i: (0, 0)),
        ],
        out_specs=pl.BlockSpec((B, D), lambda i: (0, 0)),
        out_shape=jax.ShapeDtypeStruct((B, D), jnp.float32),
        scratch_shapes=[
            pltpu.VMEM((B, D), jnp.float32),
            pltpu.VMEM((B, 1), jnp.float32),
        ],
        interpret=interpret,
    )(x, batch3, wc, w3c)


def kernel(x, batch, W1, W2, W3):
    batch3 = batch.reshape(NB, BLK, 1)
    wc = jnp.concatenate([W1.T, W2.T], axis=1).astype(jnp.bfloat16)
    return _run(x, batch3, wc, W3.T.astype(jnp.bfloat16))


# speedup vs baseline: 2.2233x; 2.2233x over previous
import functools

import jax
import jax.numpy as jnp
from jax.experimental import pallas as pl
from jax.experimental.pallas import tpu as pltpu

N = 100000
D = 128
H = 64
B = 64
BLK = 2000
NB = N // BLK


def _fused_body(x_ref, b_ref, w1t_ref, w2t_ref, w3c_ref, out_ref, zacc, dacc):
    i = pl.program_id(0)

    @pl.when(i == 0)
    def _init():
        zacc[:, :] = jnp.zeros_like(zacc)
        dacc[:, :] = jnp.zeros_like(dacc)

    f32 = jnp.float32
    xb = x_ref[:, :]
    u = jnp.tanh(jax.lax.dot_general(
        xb, w1t_ref[:, :], (((1,), (0,)), ((), ())),
        preferred_element_type=f32))
    e = jnp.exp(jax.lax.dot_general(
        xb, w2t_ref[:, :], (((1,), (0,)), ((), ())),
        preferred_element_type=f32))
    v = e / jnp.sum(e, axis=1, keepdims=True)
    g = u * v
    alpha = jax.lax.dot_general(
        g, w3c_ref[:, :], (((1,), (0,)), ((), ())),
        preferred_element_type=f32)
    w = jnp.exp(alpha)

    ids = b_ref[0]
    seg = jax.lax.broadcasted_iota(jnp.int32, (BLK, B), 1)
    m = jnp.where(ids == seg, w, 0.0)

    zacc[:, :] += jax.lax.dot_general(
        m, xb, (((0,), (0,)), ((), ())),
        preferred_element_type=f32)
    dacc[:, :] += jax.lax.dot_general(
        m, jnp.ones((BLK, 1), f32), (((0,), (0,)), ((), ())),
        preferred_element_type=f32)

    @pl.when(i == NB - 1)
    def _emit():
        out_ref[:, :] = zacc[:, :] / jnp.maximum(dacc[:, :], 1e-30)


@functools.partial(jax.jit, static_argnames=("interpret",))
def _run(x, batch3, w1t, w2t, w3c, interpret=False):
    return pl.pallas_call(
        _fused_body,
        grid=(NB,),
        in_specs=[
            pl.BlockSpec((BLK, D), lambda i: (i, 0)),
            pl.BlockSpec((1, BLK, 1), lambda i: (i, 0, 0)),
            pl.BlockSpec((D, H), lambda i: (0, 0)),
            pl.BlockSpec((D, H), lambda i: (0, 0)),
            pl.BlockSpec((H, 1), lambda i: (0, 0)),
        ],
        out_specs=pl.BlockSpec((B, D), lambda i: (0, 0)),
        out_shape=jax.ShapeDtypeStruct((B, D), jnp.float32),
        scratch_shapes=[
            pltpu.VMEM((B, D), jnp.float32),
            pltpu.VMEM((B, 1), jnp.float32),
        ],
        interpret=interpret,
    )(x, batch3, w1t, w2t, w3c)


def kernel(x, batch, W1, W2, W3):
    batch3 = batch.reshape(NB, BLK, 1)
    return _run(x, batch3, W1.T, W2.T, W3.T)
